# baseline (speedup 1.0000x reference)
import functools

import jax
import jax.numpy as jnp
from jax import lax
from jax.experimental import pallas as pl
from jax.experimental.pallas import tpu as pltpu
from jax.experimental.pallas import tpu_sc as plsc

_TABLE_ROWS = 8192
_D = 1024
_B_TOTAL = 8192
_NC = 2
_NS = 16
_NW = _NC * _NS
_B_PER_W = _B_TOTAL // _NW
_CHUNK = 32
_N_CHUNKS = _B_PER_W // _CHUNK
_NBUF = 3


def _gather_sc(table, idx):
  mesh = plsc.VectorSubcoreMesh(core_axis_name="c", subcore_axis_name="s")

  @functools.partial(
      pl.kernel,
      mesh=mesh,
      out_type=jax.ShapeDtypeStruct((_NW, _N_CHUNKS, _CHUNK, _D),
                                    jnp.float32),
      scratch_types=[
          pltpu.VMEM((_N_CHUNKS, _CHUNK), jnp.int32),
      ] + [pltpu.VMEM((_CHUNK, _D), jnp.float32) for _ in range(_NBUF)]
        + [pltpu.SemaphoreType.DMA for _ in range(2 * _NBUF)],
  )
  def k(table_hbm, idx_hbm, out_hbm, idx_v, *rest):
    bufs = rest[:_NBUF]
    sgs = rest[_NBUF:2 * _NBUF]
    sws = rest[2 * _NBUF:]
    wid = lax.axis_index("s") * _NC + lax.axis_index("c")
    pltpu.sync_copy(idx_hbm.at[wid], idx_v)

    gathers = [None] * _N_CHUNKS
    writes = [None] * _N_CHUNKS
    for c in range(min(_NBUF - 1, _N_CHUNKS)):
      gathers[c] = pltpu.async_copy(
          table_hbm.at[idx_v.at[c]], bufs[c % _NBUF], sgs[c % _NBUF])
    for c in range(_N_CHUNKS):
      gathers[c].wait()
      writes[c] = pltpu.async_copy(bufs[c % _NBUF], out_hbm.at[wid, c],
                                   sws[c % _NBUF])
      nxt = c + _NBUF - 1
      if nxt < _N_CHUNKS:
        if c >= 1:
          writes[c - 1].wait()
        gathers[nxt] = pltpu.async_copy(
            table_hbm.at[idx_v.at[nxt]], bufs[nxt % _NBUF], sgs[nxt % _NBUF])
    for c in range(max(0, _N_CHUNKS - _NBUF), _N_CHUNKS):
      writes[c].wait()

  return k(table, idx)


def kernel(position_ids, pos_emb):
  batch, seq = position_ids.shape
  table = pos_emb.reshape(_TABLE_ROWS, _D)
  idx = position_ids.reshape(_NW, _N_CHUNKS, _CHUNK).astype(jnp.int32)
  out = _gather_sc(table, idx)
  return out.reshape(1, batch, seq, _D)


# speedup vs baseline: 1.0175x; 1.0175x over previous
import functools

import jax
import jax.numpy as jnp
from jax import lax
from jax.experimental import pallas as pl
from jax.experimental.pallas import tpu as pltpu
from jax.experimental.pallas import tpu_sc as plsc

_TABLE_ROWS = 8192
_D = 1024
_B_TOTAL = 8192
_NC = 2
_NS = 16
_NW = _NC * _NS
_B_PER_W = _B_TOTAL // _NW
_CHUNK = 8
_N_CHUNKS = _B_PER_W // _CHUNK
_NBUF = 8


def _gather_sc(table, idx):
  mesh = plsc.VectorSubcoreMesh(core_axis_name="c", subcore_axis_name="s")

  @functools.partial(
      pl.kernel,
      mesh=mesh,
      out_type=jax.ShapeDtypeStruct((_NW, _N_CHUNKS, _CHUNK, _D),
                                    jnp.float32),
      scratch_types=[
          pltpu.VMEM((_N_CHUNKS, _CHUNK), jnp.int32),
      ] + [pltpu.VMEM((_CHUNK, _D), jnp.float32) for _ in range(_NBUF)]
        + [pltpu.SemaphoreType.DMA for _ in range(2 * _NBUF)],
  )
  def k(table_hbm, idx_hbm, out_hbm, idx_v, *rest):
    bufs = rest[:_NBUF]
    sgs = rest[_NBUF:2 * _NBUF]
    sws = rest[2 * _NBUF:]
    wid = lax.axis_index("s") * _NC + lax.axis_index("c")
    pltpu.sync_copy(idx_hbm.at[wid], idx_v)

    gathers = [None] * _N_CHUNKS
    writes = [None] * _N_CHUNKS
    for c in range(min(_NBUF - 1, _N_CHUNKS)):
      gathers[c] = pltpu.async_copy(
          table_hbm.at[idx_v.at[c]], bufs[c % _NBUF], sgs[c % _NBUF])
    for c in range(_N_CHUNKS):
      gathers[c].wait()
      writes[c] = pltpu.async_copy(bufs[c % _NBUF], out_hbm.at[wid, c],
                                   sws[c % _NBUF])
      nxt = c + _NBUF - 1
      if nxt < _N_CHUNKS:
        if c >= 1:
          writes[c - 1].wait()
        gathers[nxt] = pltpu.async_copy(
            table_hbm.at[idx_v.at[nxt]], bufs[nxt % _NBUF], sgs[nxt % _NBUF])
    for c in range(max(0, _N_CHUNKS - _NBUF), _N_CHUNKS):
      writes[c].wait()

  return k(table, idx)


def kernel(position_ids, pos_emb):
  batch, seq = position_ids.shape
  table = pos_emb.reshape(_TABLE_ROWS, _D)
  idx = position_ids.reshape(_NW, _N_CHUNKS, _CHUNK).astype(jnp.int32)
  out = _gather_sc(table, idx)
  return out.reshape(1, batch, seq, _D)


# speedup vs baseline: 1.0241x; 1.0066x over previous
import functools

import jax
import jax.numpy as jnp
from jax import lax
from jax.experimental import pallas as pl
from jax.experimental.pallas import tpu as pltpu
from jax.experimental.pallas import tpu_sc as plsc

_TABLE_ROWS = 8192
_D = 1024
_B_TOTAL = 8192
_NC = 2
_NS = 16
_NW = _NC * _NS
_B_PER_W = _B_TOTAL // _NW
_CHUNK = 8
_N_CHUNKS = _B_PER_W // _CHUNK
_NBUF = 8


def _gather_sc(table, idx):
  mesh = plsc.VectorSubcoreMesh(core_axis_name="c", subcore_axis_name="s")

  @functools.partial(
      pl.kernel,
      mesh=mesh,
      out_type=jax.ShapeDtypeStruct((_B_TOTAL, _D), jnp.float32),
      scratch_types=[
          pltpu.VMEM((_B_PER_W,), jnp.int32),
      ] + [pltpu.VMEM((_CHUNK, _D), jnp.float32) for _ in range(_NBUF)]
        + [pltpu.SemaphoreType.DMA for _ in range(2 * _NBUF)],
  )
  def k(table_hbm, idx_hbm, out_hbm, idx_v, *rest):
    bufs = rest[:_NBUF]
    sgs = rest[_NBUF:2 * _NBUF]
    sws = rest[2 * _NBUF:]
    wid = lax.axis_index("s") * _NC + lax.axis_index("c")
    base = wid * _B_PER_W
    pltpu.sync_copy(idx_hbm.at[pl.ds(base, _B_PER_W)], idx_v)

    gathers = [None] * _N_CHUNKS
    writes = [None] * _N_CHUNKS
    for c in range(min(_NBUF - 1, _N_CHUNKS)):
      gathers[c] = pltpu.async_copy(
          table_hbm.at[idx_v.at[pl.ds(c * _CHUNK, _CHUNK)]],
          bufs[c % _NBUF], sgs[c % _NBUF])
    for c in range(_N_CHUNKS):
      gathers[c].wait()
      writes[c] = pltpu.async_copy(
          bufs[c % _NBUF], out_hbm.at[pl.ds(base + c * _CHUNK, _CHUNK)],
          sws[c % _NBUF])
      nxt = c + _NBUF - 1
      if nxt < _N_CHUNKS:
        if c >= 1:
          writes[c - 1].wait()
        gathers[nxt] = pltpu.async_copy(
            table_hbm.at[idx_v.at[pl.ds(nxt * _CHUNK, _CHUNK)]],
            bufs[nxt % _NBUF], sgs[nxt % _NBUF])
    for c in range(max(0, _N_CHUNKS - _NBUF), _N_CHUNKS):
      writes[c].wait()

  return k(table, idx)


def kernel(position_ids, pos_emb):
  batch, seq = position_ids.shape
  table = pos_emb.reshape(_TABLE_ROWS, _D)
  idx = position_ids.reshape(_B_TOTAL).astype(jnp.int32)
  out = _gather_sc(table, idx)
  return out.reshape(1, batch, seq, _D)


# speedup vs baseline: 1.0341x; 1.0097x over previous
import functools

import jax
import jax.numpy as jnp
from jax import lax
from jax.experimental import pallas as pl
from jax.experimental.pallas import tpu as pltpu
from jax.experimental.pallas import tpu_sc as plsc

_TABLE_ROWS = 8192
_D = 1024
_B_TOTAL = 8192
_NC = 2
_NS = 16
_NW = _NC * _NS
_B_PER_W = _B_TOTAL // _NW
_CHUNK = 8
_N_CHUNKS = _B_PER_W // _CHUNK
_NBUF = 15
_W_PER_B = 8


def _gather_sc(table, idx):
  mesh = plsc.VectorSubcoreMesh(core_axis_name="c", subcore_axis_name="s")

  @functools.partial(
      pl.kernel,
      mesh=mesh,
      out_type=jax.ShapeDtypeStruct((_B_TOTAL, _D), jnp.float32),
      scratch_types=[
          pltpu.VMEM((_B_PER_W,), jnp.int32),
      ] + [pltpu.VMEM((_CHUNK, _D), jnp.float32) for _ in range(_NBUF)]
        + [pltpu.SemaphoreType.DMA for _ in range(2 * _NBUF)],
  )
  def k(table_hbm, idx_hbm, out_hbm, idx_v, *rest):
    bufs = rest[:_NBUF]
    sgs = rest[_NBUF:2 * _NBUF]
    sws = rest[2 * _NBUF:]
    wid = lax.axis_index("s") * _NC + lax.axis_index("c")
    base = wid * _B_PER_W
    pltpu.sync_copy(
        idx_hbm.at[wid // _W_PER_B,
                   pl.ds((wid % _W_PER_B) * _B_PER_W, _B_PER_W)], idx_v)

    gathers = [None] * _N_CHUNKS
    writes = [None] * _N_CHUNKS
    for c in range(min(_NBUF - 1, _N_CHUNKS)):
      gathers[c] = pltpu.async_copy(
          table_hbm.at[idx_v.at[pl.ds(c * _CHUNK, _CHUNK)]],
          bufs[c % _NBUF], sgs[c % _NBUF])
    for c in range(_N_CHUNKS):
      gathers[c].wait()
      writes[c] = pltpu.async_copy(
          bufs[c % _NBUF], out_hbm.at[pl.ds(base + c * _CHUNK, _CHUNK)],
          sws[c % _NBUF])
      nxt = c + _NBUF - 1
      if nxt < _N_CHUNKS:
        if c >= 1:
          writes[c - 1].wait()
        gathers[nxt] = pltpu.async_copy(
            table_hbm.at[idx_v.at[pl.ds(nxt * _CHUNK, _CHUNK)]],
            bufs[nxt % _NBUF], sgs[nxt % _NBUF])
    for c in range(max(0, _N_CHUNKS - _NBUF), _N_CHUNKS):
      writes[c].wait()

  return k(table, idx)


def kernel(position_ids, pos_emb):
  batch, seq = position_ids.shape
  table = pos_emb.reshape(_TABLE_ROWS, _D)
  out = _gather_sc(table, position_ids)
  return out.reshape(1, batch, seq, _D)
